# baseline (speedup 1.0000x reference)
import functools

import jax
import jax.numpy as jnp
from jax import lax
from jax.experimental import pallas as pl
from jax.experimental.pallas import tpu as pltpu
from jax.experimental.pallas import tpu_sc as plsc

_B, _C, _K, _HW = 16, 256, 128, 1024


def _tc_body(q_ref, w_ref, idx_ref, loss_ref):
    q = q_ref[0]
    w = w_ref[...]

    ab = lax.dot_general(w, q, (((1,), (0,)), ((), ())),
                         preferred_element_type=jnp.float32)
    b2 = jnp.sum(w * w, axis=1, keepdims=True)
    a2 = jnp.sum(q * q, axis=0, keepdims=True)
    d2 = jnp.maximum(a2 + b2 - 2.0 * ab, 0.0)

    m = jnp.min(d2, axis=0, keepdims=True)
    kio = lax.broadcasted_iota(jnp.int32, (_K, _HW), 0)
    idx = jnp.min(jnp.where(d2 == m, kio, jnp.int32(_K)),
                  axis=0, keepdims=True)
    idx_ref[0] = idx
    loss_ref[0, 0, 0] = jnp.sum(m)


def _tc_stage(q, W):
    return pl.pallas_call(
        _tc_body,
        grid=(_B,),
        in_specs=[
            pl.BlockSpec((1, _C, _HW), lambda b: (b, 0, 0)),
            pl.BlockSpec((_K, _C), lambda b: (0, 0)),
        ],
        out_specs=[
            pl.BlockSpec((1, 1, _HW), lambda b: (b, 0, 0)),
            pl.BlockSpec((1, 1, 1), lambda b: (b, 0, 0), memory_space=pltpu.SMEM),
        ],
        out_shape=[
            jax.ShapeDtypeStruct((_B, 1, _HW), jnp.int32),
            jax.ShapeDtypeStruct((_B, 1, 1), jnp.float32),
        ],
    )(q, W)


_CPW = _C // 32


def _sc_gather(wflat, idx):
    mesh = plsc.VectorSubcoreMesh(core_axis_name="c", subcore_axis_name="s")

    @functools.partial(
        pl.kernel, mesh=mesh,
        compiler_params=pltpu.CompilerParams(needs_layout_passes=False),
        out_type=jax.ShapeDtypeStruct((_B, _C, _HW), jnp.float32),
        scratch_types=[
            pltpu.VMEM((_K * _C,), jnp.float32),
            pltpu.VMEM((_HW,), jnp.int32),
            pltpu.VMEM((_HW,), jnp.int32),
            pltpu.VMEM((_CPW, _HW), jnp.float32),
            pltpu.VMEM((_CPW, _HW), jnp.float32),
            pltpu.SemaphoreType.DMA,
            pltpu.SemaphoreType.DMA,
        ],
    )
    def body(w_hbm, idx_hbm, out_hbm, w_v, row_v, sidx_v, buf_a, buf_b, sem_a, sem_b):
        wid = lax.axis_index("s") * 2 + lax.axis_index("c")
        c0 = wid * _CPW
        pltpu.sync_copy(w_hbm, w_v)

        bufs = (buf_a, buf_b)
        sems = (sem_a, sem_b)
        copies = [None, None]
        for b in range(_B):
            pltpu.sync_copy(idx_hbm.at[b], row_v)

            def build(j, _):
                p = lax.broadcasted_iota(jnp.int32, (16,), 0) + j * 16
                pt = ((p & 31) << 5) | (p >> 5)
                v = plsc.load_gather(row_v, [pt])
                sidx_v[pl.ds(j * 16, 16)] = v << 8
                return 0

            lax.fori_loop(0, _HW // 16, build, 0, unroll=4)

            slot = b & 1
            if copies[slot] is not None:
                copies[slot].wait()
            buf = bufs[slot]
            for ci in range(_CPW):
                c = c0 + ci

                def gather(j, _):
                    fi = sidx_v[pl.ds(j * 16, 16)] + c
                    buf[ci, pl.ds(j * 16, 16)] = plsc.load_gather(w_v, [fi])
                    return 0

                lax.fori_loop(0, _HW // 16, gather, 0, unroll=4)

            cp = pltpu.make_async_copy(
                buf, out_hbm.at[b, pl.ds(c0, _CPW)], sems[slot])
            cp.start()
            copies[slot] = cp
        for cp in copies:
            if cp is not None:
                cp.wait()

    return body(wflat, idx)


def kernel(pre_quantized, W):
    q = pre_quantized.reshape(_B, _C, _HW)
    idx, loss_parts = _tc_stage(q, W)
    out = _sc_gather(W.reshape(_K * _C), idx.reshape(_B, _HW))
    loss = jnp.sum(loss_parts) * (1.25 / (_B * _HW * _C))
    return out.reshape(_B, _C, 32, 32), loss


# speedup vs baseline: 1.5387x; 1.5387x over previous
import functools

import jax
import jax.numpy as jnp
from jax import lax
from jax.experimental import pallas as pl
from jax.experimental.pallas import tpu as pltpu
from jax.experimental.pallas import tpu_sc as plsc

_B, _C, _K, _HW = 16, 256, 128, 1024


def _tc_body(q_ref, w_ref, p_ref, idx_ref, loss_ref):
    q = q_ref[0]
    w = w_ref[...]

    ab = lax.dot_general(w, q, (((1,), (0,)), ((), ())),
                         preferred_element_type=jnp.float32)
    b2 = jnp.sum(w * w, axis=1, keepdims=True)
    a2 = jnp.sum(q * q, axis=0, keepdims=True)
    d2 = jnp.maximum(a2 + b2 - 2.0 * ab, 0.0)

    m = jnp.min(d2, axis=0, keepdims=True)
    kio = lax.broadcasted_iota(jnp.int32, (_K, _HW), 0)
    idx = jnp.min(jnp.where(d2 == m, kio, jnp.int32(_K)),
                  axis=0, keepdims=True)
    idxt = lax.dot_general(idx.astype(jnp.float32), p_ref[...],
                           (((1,), (0,)), ((), ())),
                           precision=lax.Precision.HIGHEST,
                           preferred_element_type=jnp.float32)
    idx_ref[0] = idxt.astype(jnp.int32)
    loss_ref[0, 0, 0] = jnp.sum(m)


def _hw_swap_perm():
    p = jnp.arange(_HW, dtype=jnp.int32)
    pt = ((p & 31) << 5) | (p >> 5)
    return (pt[:, None] == p[None, :]).astype(jnp.float32)


def _tc_stage(q, W):
    return pl.pallas_call(
        _tc_body,
        grid=(_B,),
        in_specs=[
            pl.BlockSpec((1, _C, _HW), lambda b: (b, 0, 0)),
            pl.BlockSpec((_K, _C), lambda b: (0, 0)),
            pl.BlockSpec((_HW, _HW), lambda b: (0, 0)),
        ],
        out_specs=[
            pl.BlockSpec((1, 1, _HW), lambda b: (b, 0, 0)),
            pl.BlockSpec((1, 1, 1), lambda b: (b, 0, 0), memory_space=pltpu.SMEM),
        ],
        out_shape=[
            jax.ShapeDtypeStruct((_B, 1, _HW), jnp.int32),
            jax.ShapeDtypeStruct((_B, 1, 1), jnp.float32),
        ],
    )(q, W, _hw_swap_perm())


_CPW = _C // 32


def _sc_gather(wtflat, idx):
    mesh = plsc.VectorSubcoreMesh(core_axis_name="c", subcore_axis_name="s")

    @functools.partial(
        pl.kernel, mesh=mesh,
        compiler_params=pltpu.CompilerParams(needs_layout_passes=False),
        out_type=jax.ShapeDtypeStruct((_B, _C, _HW), jnp.float32),
        scratch_types=[
            pltpu.VMEM((_C * _K,), jnp.float32),
            pltpu.VMEM((_HW,), jnp.int32),
            pltpu.VMEM((_CPW, _HW), jnp.float32),
            pltpu.VMEM((_CPW, _HW), jnp.float32),
            pltpu.SemaphoreType.DMA,
            pltpu.SemaphoreType.DMA,
        ],
    )
    def body(w_hbm, idx_hbm, out_hbm, w_v, row_v, buf_a, buf_b, sem_a, sem_b):
        wid = lax.axis_index("s") * 2 + lax.axis_index("c")
        c0 = wid * _CPW
        pltpu.sync_copy(w_hbm, w_v)

        bufs = (buf_a, buf_b)
        sems = (sem_a, sem_b)
        copies = [None, None]
        for b in range(_B):
            pltpu.sync_copy(idx_hbm.at[b], row_v)

            slot = b & 1
            if copies[slot] is not None:
                copies[slot].wait()
            buf = bufs[slot]

            def gather(j, _):
                v = row_v[pl.ds(j * 16, 16)]
                for ci in range(_CPW):
                    fi = v + (c0 + ci) * _K
                    buf[ci, pl.ds(j * 16, 16)] = plsc.load_gather(w_v, [fi])
                return 0

            lax.fori_loop(0, _HW // 16, gather, 0, unroll=2)

            cp = pltpu.make_async_copy(
                buf, out_hbm.at[b, pl.ds(c0, _CPW)], sems[slot])
            cp.start()
            copies[slot] = cp
        for cp in copies:
            if cp is not None:
                cp.wait()

    return body(wtflat, idx)


def kernel(pre_quantized, W):
    q = pre_quantized.reshape(_B, _C, _HW)
    idx, loss_parts = _tc_stage(q, W)
    out = _sc_gather(W.T.reshape(_C * _K), idx.reshape(_B, _HW))
    loss = jnp.sum(loss_parts) * (1.25 / (_B * _HW * _C))
    return out.reshape(_B, _C, 32, 32), loss


# speedup vs baseline: 1.9242x; 1.2505x over previous
import functools

import jax
import jax.numpy as jnp
from jax import lax
from jax.experimental import pallas as pl
from jax.experimental.pallas import tpu as pltpu
from jax.experimental.pallas import tpu_sc as plsc

_B, _C, _K, _HW = 16, 256, 128, 1024


def _tc_body(q_ref, w_ref, p_ref, idx_ref, loss_ref):
    q = q_ref[0]
    w = w_ref[...]

    ab = lax.dot_general(w, q, (((1,), (0,)), ((), ())),
                         preferred_element_type=jnp.float32)
    b2 = jnp.sum(w * w, axis=1, keepdims=True)
    a2 = jnp.sum(q * q, axis=0, keepdims=True)
    d2 = jnp.maximum(a2 + b2 - 2.0 * ab, 0.0)

    m = jnp.min(d2, axis=0, keepdims=True)
    kio = lax.broadcasted_iota(jnp.int32, (_K, _HW), 0)
    idx = jnp.min(jnp.where(d2 == m, kio, jnp.int32(_K)),
                  axis=0, keepdims=True)
    idxt = lax.dot_general(idx.astype(jnp.float32), p_ref[...],
                           (((1,), (0,)), ((), ())),
                           precision=lax.Precision.HIGHEST,
                           preferred_element_type=jnp.float32)
    idx_ref[0] = idxt.astype(jnp.int32)
    loss_ref[0, 0, 0] = jnp.sum(m)


def _hw_swap_perm():
    p = jnp.arange(_HW, dtype=jnp.int32)
    pt = ((p & 31) << 5) | (p >> 5)
    return (pt[:, None] == p[None, :]).astype(jnp.float32)


def _tc_stage(q, W):
    return pl.pallas_call(
        _tc_body,
        grid=(_B,),
        in_specs=[
            pl.BlockSpec((1, _C, _HW), lambda b: (b, 0, 0)),
            pl.BlockSpec((_K, _C), lambda b: (0, 0)),
            pl.BlockSpec((_HW, _HW), lambda b: (0, 0)),
        ],
        out_specs=[
            pl.BlockSpec((1, 1, _HW), lambda b: (b, 0, 0)),
            pl.BlockSpec((1, 1, 1), lambda b: (b, 0, 0), memory_space=pltpu.SMEM),
        ],
        out_shape=[
            jax.ShapeDtypeStruct((_B, 1, _HW), jnp.int32),
            jax.ShapeDtypeStruct((_B, 1, 1), jnp.float32),
        ],
    )(q, W, _hw_swap_perm())


_CPW = _C // 32


def _sc_gather(wt, idx):
    mesh = plsc.VectorSubcoreMesh(core_axis_name="c", subcore_axis_name="s")

    @functools.partial(
        pl.kernel, mesh=mesh,
        compiler_params=pltpu.CompilerParams(needs_layout_passes=False),
        out_type=jax.ShapeDtypeStruct((_B, _C, _HW), jnp.float32),
        scratch_types=[
            pltpu.VMEM((_C, _K), jnp.float32),
            pltpu.VMEM((_B, _HW), jnp.int32),
            pltpu.VMEM((_CPW, _HW), jnp.float32),
            pltpu.VMEM((_CPW, _HW), jnp.float32),
            pltpu.SemaphoreType.DMA,
            pltpu.SemaphoreType.DMA,
        ],
    )
    def body(w_hbm, idx_hbm, out_hbm, w_v, idx_v, buf_a, buf_b, sem_a, sem_b):
        wid = lax.axis_index("s") * 2 + lax.axis_index("c")
        c0 = wid * _CPW
        pltpu.sync_copy(w_hbm, w_v)
        pltpu.sync_copy(idx_hbm, idx_v)

        bufs = (buf_a, buf_b)
        sems = (sem_a, sem_b)
        copies = [None, None]
        for b in range(_B):
            slot = b & 1
            if copies[slot] is not None:
                copies[slot].wait()
            buf = bufs[slot]

            def gather(j, _):
                v = idx_v[b, pl.ds(j * 16, 16)]
                vals = [plsc.load_gather(w_v.at[c0 + ci], [v])
                        for ci in range(_CPW)]
                for ci in range(_CPW):
                    buf[ci, pl.ds(j * 16, 16)] = vals[ci]
                return 0

            lax.fori_loop(0, _HW // 16, gather, 0, unroll=4)

            cp = pltpu.make_async_copy(
                buf, out_hbm.at[b, pl.ds(c0, _CPW)], sems[slot])
            cp.start()
            copies[slot] = cp
        for cp in copies:
            if cp is not None:
                cp.wait()

    return body(wt, idx)


def kernel(pre_quantized, W):
    q = pre_quantized.reshape(_B, _C, _HW)
    idx, loss_parts = _tc_stage(q, W)
    out = _sc_gather(W.T, idx.reshape(_B, _HW))
    loss = jnp.sum(loss_parts) * (1.25 / (_B * _HW * _C))
    return out.reshape(_B, _C, 32, 32), loss


# speedup vs baseline: 2.3047x; 1.1978x over previous
import functools

import jax
import jax.numpy as jnp
from jax import lax
from jax.experimental import pallas as pl
from jax.experimental.pallas import tpu as pltpu
from jax.experimental.pallas import tpu_sc as plsc

_B, _C, _K, _HW = 16, 256, 128, 1024


def _tc_body(q_ref, w_ref, p_ref, idx_ref, loss_ref):
    q = q_ref[0]
    w = w_ref[...]

    ab = lax.dot_general(w, q, (((1,), (0,)), ((), ())),
                         preferred_element_type=jnp.float32)
    b2 = jnp.sum(w * w, axis=1, keepdims=True)
    a2 = jnp.sum(q * q, axis=0, keepdims=True)
    d2 = jnp.maximum(a2 + b2 - 2.0 * ab, 0.0)

    m = jnp.min(d2, axis=0, keepdims=True)
    kio = lax.broadcasted_iota(jnp.int32, (_K, _HW), 0)
    idx = jnp.min(jnp.where(d2 == m, kio, jnp.int32(_K)),
                  axis=0, keepdims=True)
    idxt = lax.dot_general(idx.astype(jnp.bfloat16), p_ref[...],
                           (((1,), (0,)), ((), ())),
                           preferred_element_type=jnp.float32)
    idx_ref[0] = idxt.astype(jnp.int32)
    loss_ref[0, 0, 0] = jnp.sum(m)


def _hw_swap_perm():
    p = jnp.arange(_HW, dtype=jnp.int32)
    pt = ((p & 31) << 5) | (p >> 5)
    return (pt[:, None] == p[None, :]).astype(jnp.bfloat16)


def _tc_stage(q, W):
    return pl.pallas_call(
        _tc_body,
        grid=(_B,),
        in_specs=[
            pl.BlockSpec((1, _C, _HW), lambda b: (b, 0, 0)),
            pl.BlockSpec((_K, _C), lambda b: (0, 0)),
            pl.BlockSpec((_HW, _HW), lambda b: (0, 0)),
        ],
        out_specs=[
            pl.BlockSpec((1, 1, _HW), lambda b: (b, 0, 0)),
            pl.BlockSpec((1, 1, 1), lambda b: (b, 0, 0), memory_space=pltpu.SMEM),
        ],
        out_shape=[
            jax.ShapeDtypeStruct((_B, 1, _HW), jnp.int32),
            jax.ShapeDtypeStruct((_B, 1, 1), jnp.float32),
        ],
    )(q, W, _hw_swap_perm())


_CPW = _C // 32


def _sc_gather(wt, idx):
    mesh = plsc.VectorSubcoreMesh(core_axis_name="c", subcore_axis_name="s")

    @functools.partial(
        pl.kernel, mesh=mesh,
        compiler_params=pltpu.CompilerParams(needs_layout_passes=False),
        out_type=jax.ShapeDtypeStruct((_B, _C, _HW), jnp.float32),
        scratch_types=[
            pltpu.VMEM((_C, _K), jnp.float32),
            pltpu.VMEM((_B, _HW), jnp.int32),
            pltpu.VMEM((_CPW, _HW), jnp.float32),
            pltpu.VMEM((_CPW, _HW), jnp.float32),
            pltpu.SemaphoreType.DMA,
            pltpu.SemaphoreType.DMA,
        ],
    )
    def body(w_hbm, idx_hbm, out_hbm, w_v, idx_v, buf_a, buf_b, sem_a, sem_b):
        wid = lax.axis_index("s") * 2 + lax.axis_index("c")
        c0 = wid * _CPW
        pltpu.sync_copy(w_hbm, w_v)
        pltpu.sync_copy(idx_hbm, idx_v)

        bufs = (buf_a, buf_b)
        sems = (sem_a, sem_b)
        copies = [None, None]
        for b in range(_B):
            slot = b & 1
            if copies[slot] is not None:
                copies[slot].wait()
            buf = bufs[slot]

            def gather(j, _):
                v = idx_v[b, pl.ds(j * 16, 16)]
                vals = [plsc.load_gather(w_v.at[c0 + ci], [v])
                        for ci in range(_CPW)]
                for ci in range(_CPW):
                    buf[ci, pl.ds(j * 16, 16)] = vals[ci]
                return 0

            lax.fori_loop(0, _HW // 16, gather, 0, unroll=4)

            cp = pltpu.make_async_copy(
                buf, out_hbm.at[b, pl.ds(c0, _CPW)], sems[slot])
            cp.start()
            copies[slot] = cp
        for cp in copies:
            if cp is not None:
                cp.wait()

    return body(wt, idx)


def kernel(pre_quantized, W):
    q = pre_quantized.reshape(_B, _C, _HW)
    idx, loss_parts = _tc_stage(q, W)
    out = _sc_gather(W.T, idx.reshape(_B, _HW))
    loss = jnp.sum(loss_parts) * (1.25 / (_B * _HW * _C))
    return out.reshape(_B, _C, 32, 32), loss


# speedup vs baseline: 2.3893x; 1.0367x over previous
import functools

import jax
import jax.numpy as jnp
from jax import lax
from jax.experimental import pallas as pl
from jax.experimental.pallas import tpu as pltpu
from jax.experimental.pallas import tpu_sc as plsc

_B, _C, _K, _HW = 16, 256, 128, 1024


def _tc_body(q_ref, w_ref, idx_ref, loss_ref):
    q = q_ref[0]
    w = w_ref[...]

    ab = lax.dot_general(w, q, (((1,), (0,)), ((), ())),
                         preferred_element_type=jnp.float32)
    b2 = jnp.sum(w * w, axis=1, keepdims=True)
    a2 = jnp.sum(q * q, axis=0, keepdims=True)
    d2 = jnp.maximum(a2 + b2 - 2.0 * ab, 0.0)

    m = jnp.min(d2, axis=0, keepdims=True)
    kio = lax.broadcasted_iota(jnp.int32, (_K, _HW), 0)
    idx = jnp.min(jnp.where(d2 == m, kio, jnp.int32(_K)),
                  axis=0, keepdims=True)
    idx_ref[0] = idx
    loss_ref[0, 0, 0] = jnp.sum(m)


def _tc_stage(q, W):
    return pl.pallas_call(
        _tc_body,
        grid=(_B,),
        in_specs=[
            pl.BlockSpec((1, _C, _HW), lambda b: (b, 0, 0)),
            pl.BlockSpec((_K, _C), lambda b: (0, 0)),
        ],
        out_specs=[
            pl.BlockSpec((1, 1, _HW), lambda b: (b, 0, 0)),
            pl.BlockSpec((1, 1, 1), lambda b: (b, 0, 0), memory_space=pltpu.SMEM),
        ],
        out_shape=[
            jax.ShapeDtypeStruct((_B, 1, _HW), jnp.int32),
            jax.ShapeDtypeStruct((_B, 1, 1), jnp.float32),
        ],
    )(q, W)


_CPW = _C // 32


def _sc_gather(wt, idx):
    mesh = plsc.VectorSubcoreMesh(core_axis_name="c", subcore_axis_name="s")

    @functools.partial(
        pl.kernel, mesh=mesh,
        compiler_params=pltpu.CompilerParams(needs_layout_passes=False),
        out_type=jax.ShapeDtypeStruct((_B, _C, _HW), jnp.float32),
        scratch_types=[
            pltpu.VMEM((_C, _K), jnp.float32),
            pltpu.VMEM((_B, _HW), jnp.int32),
            pltpu.VMEM((_CPW, _HW), jnp.float32),
            pltpu.VMEM((_CPW, _HW), jnp.float32),
            pltpu.SemaphoreType.DMA,
            pltpu.SemaphoreType.DMA,
        ],
    )
    def body(w_hbm, idx_hbm, out_hbm, w_v, idx_v, buf_a, buf_b, sem_a, sem_b):
        wid = lax.axis_index("s") * 2 + lax.axis_index("c")
        c0 = wid * _CPW
        pltpu.sync_copy(w_hbm, w_v)
        pltpu.sync_copy(idx_hbm, idx_v)

        bufs = (buf_a, buf_b)
        sems = (sem_a, sem_b)
        copies = [None, None]
        for b in range(_B):
            slot = b & 1
            if copies[slot] is not None:
                copies[slot].wait()
            buf = bufs[slot]

            def gather(j, _):
                v = idx_v[b, pl.ds(j * 16, 16)]
                vals = [plsc.load_gather(w_v.at[c0 + ci], [v])
                        for ci in range(_CPW)]
                for ci in range(_CPW):
                    buf[ci, pl.ds(j * 16, 16)] = vals[ci]
                return 0

            lax.fori_loop(0, _HW // 16, gather, 0, unroll=4)

            cp = pltpu.make_async_copy(
                buf, out_hbm.at[b, pl.ds(c0, _CPW)], sems[slot])
            cp.start()
            copies[slot] = cp
        for cp in copies:
            if cp is not None:
                cp.wait()

    return body(wt, idx)


def kernel(pre_quantized, W):
    q = pre_quantized.reshape(_B, _C, _HW)
    idx, loss_parts = _tc_stage(q, W)
    idxs = idx.reshape(_B, 32, 32).swapaxes(1, 2).reshape(_B, _HW)
    out = _sc_gather(W.T, idxs)
    loss = jnp.sum(loss_parts) * (1.25 / (_B * _HW * _C))
    return out.reshape(_B, _C, 32, 32), loss
